# baseline (speedup 1.0000x reference)
import functools

import jax
import jax.numpy as jnp
from jax import lax
from jax.experimental import pallas as pl
from jax.experimental.pallas import tpu as pltpu
from jax.experimental.pallas import tpu_sc as plsc

NUM_CLASSES = 401
TOP_K = 16
THRESH = 0.05
BLOCK_ROWS = 800

_CBITS = 0x3D4CCCCD


def _topk_body(x_ref, vals_ref, idx_ref):
    x = x_ref[...]
    m = jnp.max(x, axis=-1, keepdims=True)
    e = jnp.exp(x - m)
    s = jnp.sum(e, axis=-1, keepdims=True)
    r = 1.0 / s
    p = e * r
    col = jax.lax.broadcasted_iota(jnp.int32, p.shape, 1)
    mask = (col != 0) & (p > THRESH)
    bits = jax.lax.bitcast_convert_type(e, jnp.int32)
    valpart = ((bits - _CBITS) >> 4) + 1
    key = jnp.where(mask, valpart << 9, 0) | (511 - col)
    keys = []
    for _ in range(TOP_K):
        mx = jnp.max(key, axis=-1, keepdims=True)
        keys.append(mx)
        key = jnp.where(key == mx, -1, key)
    k16 = jnp.concatenate(keys, axis=-1)
    vp = k16 >> 9
    e_rec = jax.lax.bitcast_convert_type(
        ((vp - 1) << 4) + (_CBITS + 8), jnp.float32)
    vals_ref[...] = jnp.where(vp > 0, e_rec * r, 0.0)
    idx_ref[...] = 511 - (k16 & 511)


def _run_topk(x, interpret=False):
    rows = x.shape[0]
    grid = (rows // BLOCK_ROWS,)
    return pl.pallas_call(
        _topk_body,
        grid=grid,
        in_specs=[pl.BlockSpec((BLOCK_ROWS, NUM_CLASSES), lambda i: (i, 0))],
        out_specs=[
            pl.BlockSpec((BLOCK_ROWS, TOP_K), lambda i: (i, 0)),
            pl.BlockSpec((BLOCK_ROWS, TOP_K), lambda i: (i, 0)),
        ],
        out_shape=[
            jax.ShapeDtypeStruct((rows, TOP_K), jnp.float32),
            jax.ShapeDtypeStruct((rows, TOP_K), jnp.int32),
        ],
        compiler_params=pltpu.CompilerParams(
            dimension_semantics=("arbitrary",)),
        interpret=interpret,
    )(x)


_SC_NW = 32
_SC_TOTAL = 20000 * 2048
_SC_PER_W = _SC_TOTAL // _SC_NW
_SC_CHUNK = 51200
_SC_NCHUNKS = _SC_PER_W // _SC_CHUNK


def _sc_copy_body(feat_ref, out_ref, buf0, buf1, rs0, rs1, ws0, ws1):
    wid = lax.axis_index("s") * 2 + lax.axis_index("c")
    base = wid * _SC_PER_W
    bufs = (buf0, buf1)
    rsems = (rs0, rs1)
    wsems = (ws0, ws1)

    def rd(j):
        return pltpu.make_async_copy(
            feat_ref.at[pl.ds(base + j * _SC_CHUNK, _SC_CHUNK)],
            bufs[j % 2], rsems[j % 2])

    def wr(j):
        return pltpu.make_async_copy(
            bufs[j % 2],
            out_ref.at[pl.ds(base + j * _SC_CHUNK, _SC_CHUNK)],
            wsems[j % 2])

    rd(0).start()
    for j in range(_SC_NCHUNKS):
        rd(j).wait()
        wr(j).start()
        if j + 1 < _SC_NCHUNKS:
            if j >= 1:
                wr(j - 1).wait()
            rd(j + 1).start()
    wr(_SC_NCHUNKS - 2).wait()
    wr(_SC_NCHUNKS - 1).wait()


def _sc_copy(features):
    flat = features.reshape(-1)
    mesh = plsc.VectorSubcoreMesh(core_axis_name="c", subcore_axis_name="s")
    fn = pl.kernel(
        _sc_copy_body,
        out_type=jax.ShapeDtypeStruct(flat.shape, flat.dtype),
        mesh=mesh,
        scratch_types=[
            pltpu.VMEM((_SC_CHUNK,), jnp.float32),
            pltpu.VMEM((_SC_CHUNK,), jnp.float32),
            pltpu.SemaphoreType.DMA,
            pltpu.SemaphoreType.DMA,
            pltpu.SemaphoreType.DMA,
            pltpu.SemaphoreType.DMA,
        ],
    )
    return fn(flat).reshape(features.shape)


def kernel(x, features):
    feat_out = _sc_copy(features)
    attr_scores, attr_labels = _run_topk(x)
    return attr_scores, attr_labels, feat_out
